# Initial kernel scaffold; baseline (speedup 1.0000x reference)
import jax
import jax.numpy as jnp
from jax.experimental import pallas as pl


def kernel(hidden_states, input_ids, emb_tables, Wk, Wv, key_norm_w, value_norm_w, conv_w, gate_bias):
    raise NotImplementedError("write your pallas kernel here")


# speedup vs baseline: 1.7952x; 1.7952x over previous
import functools

import jax
import jax.numpy as jnp
from jax import lax
from jax.experimental import pallas as pl
from jax.experimental.pallas import tpu as pltpu
from jax.experimental.pallas import tpu_sc as plsc

B, S, HID = 4, 4096, 1024
TOKEN_VOCAB = 10240
V = 100000
NGRAMS = [2, 3]
NH = 4
TOTAL = len(NGRAMS) * NH
HD = 128
MEMD = TOTAL * HD
K = 3
LAYER_ID = 0
HASH_SEED = 17
MOD = V - 1
EPS = 1e-6
POS = B * S
ROWS = TOTAL * POS

_MAX_INT = (1 << 31) - 1


def _hash_params(n):
    mults, offs = [], []
    for h in range(NH):
        base = HASH_SEED + 10007 * (LAYER_ID + 1) + 1543 * (n + 1) + 8191 * (h + 1)
        hm = []
        for pos in range(n):
            v = (base + 32771 * (pos + 1) + 65537 * (h + 1) * (pos + 1)) % _MAX_INT
            hm.append(v * 2 + 1)
        off = (base * 48271 + 97 * (n + h + 1)) % _MAX_INT
        mults.append(hm)
        offs.append(off)
    return mults, offs


_HP = {n: _hash_params(n) for n in NGRAMS}
_R32 = (1 << 32) % MOD
_R24 = (1 << 24) % MOD


def _hash_body(ids_ref, out_ref):
    x = ids_ref[...]
    pos_iota = lax.broadcasted_iota(jnp.int32, (B, S), 1)
    shifted = {0: x}
    for sh in (1, 2):
        shifted[sh] = jnp.concatenate(
            [jnp.zeros((B, sh), jnp.int32), x[:, : S - sh]], axis=1)
    g = 0
    for n in NGRAMS:
        mults, offs = _HP[n]
        for h in range(NH):
            x0 = jnp.zeros((B, S), jnp.int32)
            x1 = jnp.zeros((B, S), jnp.int32)
            x2 = jnp.zeros((B, S), jnp.int32)
            for p in range(n):
                m = mults[h][p]
                m0 = m & 0xFFFF
                m1 = m >> 16
                idsh = shifted[n - 1 - p]
                a = idsh * m0
                bb = idsh * m1
                a0 = a & 0xFFFF
                a1 = a >> 16
                b0 = bb & 0xFFFF
                b1 = bb >> 16
                l1t = a1 + b0
                c = l1t >> 16
                l1 = l1t & 0xFFFF
                l2 = b1 + c
                x0 = x0 ^ a0
                x1 = x1 ^ l1
                x2 = x2 ^ l2
            off = offs[h]
            o0 = off & 0xFFFF
            o1 = off >> 16
            s0t = x0 + o0
            c0 = s0t >> 16
            s0 = s0t & 0xFFFF
            s1t = x1 + o1 + c0
            c1 = s1t >> 16
            s1 = s1t & 0xFFFF
            s2 = x2 + c1
            tot = s2 * _R32 + (s1 >> 8) * _R24 + (s1 & 0xFF) * 65536 + s0
            r = jnp.remainder(tot, MOD) + 1
            flat = jnp.where(pos_iota >= n - 1, r, 0) + g * V
            out_ref[g] = flat
            g += 1


def _hash_ids(ids32):
    return pl.pallas_call(
        _hash_body,
        grid=(1,),
        in_specs=[pl.BlockSpec((B, S), lambda i: (_i32(i * 0), _i32(i * 0)))],
        out_specs=pl.BlockSpec(
            (TOTAL, B, S),
            lambda i: (_i32(i * 0), _i32(i * 0), _i32(i * 0))),
        out_shape=jax.ShapeDtypeStruct((TOTAL, B, S), jnp.int32),
    )(ids32)


_NC, _NS = 2, 16
_NW = _NC * _NS
_RPW = ROWS // _NW
_CHUNK = 128
_NCH = _RPW // _CHUNK


def _i32(x):
    return jnp.asarray(x, jnp.int32)


def _gather_body(table_hbm, idx_hbm, out_hbm, idx_v, buf_v, sem0, sem1):
    wid = _i32(lax.axis_index("s") * _NC + lax.axis_index("c"))
    base = wid * _RPW
    pltpu.sync_copy(idx_hbm.at[pl.ds(wid * _NCH, _NCH)], idx_v)
    sems = (sem0, sem1)

    def start(bslot, chunk):
        pltpu.make_async_copy(
            table_hbm.at[idx_v.at[_i32(chunk)]], buf_v.at[_i32(bslot)],
            sems[bslot]
        ).start()

    def finish(bslot, chunk):
        pltpu.make_async_copy(
            table_hbm.at[idx_v.at[_i32(chunk)]], buf_v.at[_i32(bslot)],
            sems[bslot]
        ).wait()
        pltpu.sync_copy(
            buf_v.at[_i32(bslot)],
            out_hbm.at[pl.ds(base + _i32(chunk) * _CHUNK, _CHUNK)])

    for b in range(2):
        start(b, b)

    def body(g2, carry):
        for b in range(2):
            g = g2 * 2 + b
            finish(b, g)
            start(b, g + 2)
        return carry

    lax.fori_loop(_i32(0), _i32((_NCH - 2) // 2), body, _i32(0))
    for b in range(2):
        finish(b, _NCH - 2 + b)


def _sc_gather(table_flat, idx2d):
    mesh = plsc.VectorSubcoreMesh(core_axis_name="c", subcore_axis_name="s")
    k = functools.partial(
        pl.kernel,
        out_type=jax.ShapeDtypeStruct((ROWS, HD), jnp.float32),
        mesh=mesh,
        scratch_types=[
            pltpu.VMEM((_NCH, _CHUNK), jnp.int32),
            pltpu.VMEM((2, _CHUNK, HD), jnp.float32),
            pltpu.SemaphoreType.DMA,
            pltpu.SemaphoreType.DMA,
        ],
    )(_gather_body)
    return k(table_flat, idx2d)


_TILE = 512
_SJ = S // _TILE


def _dense_body(mem_ref, hid_ref, wk_ref, wv_ref, knw_ref, vnw_ref, cw_ref,
                out_ref, carry_ref):
    j = pl.program_id(1)
    x = hid_ref[0]
    prev = jnp.where(j > 0, carry_ref[...], 0.0)
    xm1 = jnp.concatenate([prev[1:2], x[:-1]], axis=0)
    xm2 = jnp.concatenate([prev, x[:-2]], axis=0)
    carry_ref[...] = x[_TILE - 2:]
    q = x * cw_ref[2:3] + xm1 * cw_ref[1:2] + xm2 * cw_ref[0:1]

    acc_k = jnp.zeros((_TILE, HID), jnp.float32)
    acc_v = jnp.zeros((_TILE, HID), jnp.float32)
    for h in range(TOTAL):
        mh = mem_ref[h]
        wk_s = wk_ref[:, h * HD:(h + 1) * HD]
        wv_s = wv_ref[:, h * HD:(h + 1) * HD]
        acc_k = acc_k + lax.dot_general(
            mh, wk_s, (((1,), (1,)), ((), ())),
            preferred_element_type=jnp.float32)
        acc_v = acc_v + lax.dot_general(
            mh, wv_s, (((1,), (1,)), ((), ())),
            preferred_element_type=jnp.float32)
    vark = jnp.mean(acc_k * acc_k, axis=-1, keepdims=True)
    k_mem = knw_ref[...] * (acc_k * lax.rsqrt(vark + EPS))
    varv = jnp.mean(acc_v * acc_v, axis=-1, keepdims=True)
    v_mem = vnw_ref[...] * (acc_v * lax.rsqrt(varv + EPS))
    logit = (jnp.sum(q * k_mem, axis=-1, keepdims=True) * (1.0 / 32.0)
             + cw_ref[K:K + 1, 0:1])
    gate = jax.nn.sigmoid(logit)
    out_ref[0] = gate * v_mem


def _dense(mem8, hidden, Wk, Wv, knw, vnw, cw):
    grid = (B, _SJ)
    return pl.pallas_call(
        _dense_body,
        grid=grid,
        in_specs=[
            pl.BlockSpec((TOTAL, _TILE, HD),
                         lambda b, j: (_i32(b * 0), _i32(b * _SJ + j),
                                       _i32(b * 0))),
            pl.BlockSpec((1, _TILE, HID),
                         lambda b, j: (_i32(b), _i32(j), _i32(b * 0))),
            pl.BlockSpec((HID, MEMD),
                         lambda b, j: (_i32(b * 0), _i32(b * 0))),
            pl.BlockSpec((HID, MEMD),
                         lambda b, j: (_i32(b * 0), _i32(b * 0))),
            pl.BlockSpec((1, HID), lambda b, j: (_i32(b * 0), _i32(b * 0))),
            pl.BlockSpec((1, HID), lambda b, j: (_i32(b * 0), _i32(b * 0))),
            pl.BlockSpec((K + 1, HID),
                         lambda b, j: (_i32(b * 0), _i32(b * 0))),
        ],
        out_specs=pl.BlockSpec((1, _TILE, HID),
                               lambda b, j: (_i32(b), _i32(j), _i32(b * 0))),
        out_shape=jax.ShapeDtypeStruct((B, S, HID), jnp.float32),
        scratch_shapes=[pltpu.VMEM((2, HID), jnp.float32)],
        compiler_params=pltpu.CompilerParams(
            dimension_semantics=("arbitrary", "arbitrary")),
    )(mem8, hidden, Wk, Wv, knw, vnw, cw)


def kernel(hidden_states, input_ids, emb_tables, Wk, Wv, key_norm_w,
           value_norm_w, conv_w, gate_bias):
    ids32 = input_ids.astype(jnp.int32)
    idxs = _hash_ids(ids32)
    idx2d = idxs.reshape(_NW * _NCH, _CHUNK)
    table_flat = emb_tables.reshape(TOTAL * V, HD)
    mem = _sc_gather(table_flat, idx2d)
    mem8 = mem.reshape(TOTAL, POS, HD)
    cwgb = jnp.concatenate(
        [conv_w.reshape(HID, K).T,
         jnp.broadcast_to(jnp.reshape(gate_bias, (1, 1)), (1, HID))], axis=0)
    out = _dense(
        mem8,
        hidden_states,
        Wk,
        Wv,
        key_norm_w.reshape(1, HID),
        value_norm_w.reshape(1, HID),
        cwgb,
    )
    return out.astype(jnp.float64)


# speedup vs baseline: 1.7961x; 1.0005x over previous
import functools

import jax
import jax.numpy as jnp
from jax import lax
from jax.experimental import pallas as pl
from jax.experimental.pallas import tpu as pltpu
from jax.experimental.pallas import tpu_sc as plsc

B, S, HID = 4, 4096, 1024
TOKEN_VOCAB = 10240
V = 100000
NGRAMS = [2, 3]
NH = 4
TOTAL = len(NGRAMS) * NH
HD = 128
MEMD = TOTAL * HD
K = 3
LAYER_ID = 0
HASH_SEED = 17
MOD = V - 1
EPS = 1e-6
POS = B * S
ROWS = TOTAL * POS

_MAX_INT = (1 << 31) - 1


def _hash_params(n):
    mults, offs = [], []
    for h in range(NH):
        base = HASH_SEED + 10007 * (LAYER_ID + 1) + 1543 * (n + 1) + 8191 * (h + 1)
        hm = []
        for pos in range(n):
            v = (base + 32771 * (pos + 1) + 65537 * (h + 1) * (pos + 1)) % _MAX_INT
            hm.append(v * 2 + 1)
        off = (base * 48271 + 97 * (n + h + 1)) % _MAX_INT
        mults.append(hm)
        offs.append(off)
    return mults, offs


_HP = {n: _hash_params(n) for n in NGRAMS}
_R32 = (1 << 32) % MOD
_R24 = (1 << 24) % MOD


def _hash_body(ids_ref, out_ref):
    x = ids_ref[...]
    pos_iota = lax.broadcasted_iota(jnp.int32, (B, S), 1)
    shifted = {0: x}
    for sh in (1, 2):
        shifted[sh] = jnp.concatenate(
            [jnp.zeros((B, sh), jnp.int32), x[:, : S - sh]], axis=1)
    g = 0
    for n in NGRAMS:
        mults, offs = _HP[n]
        for h in range(NH):
            x0 = jnp.zeros((B, S), jnp.int32)
            x1 = jnp.zeros((B, S), jnp.int32)
            x2 = jnp.zeros((B, S), jnp.int32)
            for p in range(n):
                m = mults[h][p]
                m0 = m & 0xFFFF
                m1 = m >> 16
                idsh = shifted[n - 1 - p]
                a = idsh * m0
                bb = idsh * m1
                a0 = a & 0xFFFF
                a1 = a >> 16
                b0 = bb & 0xFFFF
                b1 = bb >> 16
                l1t = a1 + b0
                c = l1t >> 16
                l1 = l1t & 0xFFFF
                l2 = b1 + c
                x0 = x0 ^ a0
                x1 = x1 ^ l1
                x2 = x2 ^ l2
            off = offs[h]
            o0 = off & 0xFFFF
            o1 = off >> 16
            s0t = x0 + o0
            c0 = s0t >> 16
            s0 = s0t & 0xFFFF
            s1t = x1 + o1 + c0
            c1 = s1t >> 16
            s1 = s1t & 0xFFFF
            s2 = x2 + c1
            tot = s2 * _R32 + (s1 >> 8) * _R24 + (s1 & 0xFF) * 65536 + s0
            q = (tot.astype(jnp.float32) * (1.0 / MOD)).astype(jnp.int32)
            r = tot - q * MOD
            r = jnp.where(r < 0, r + MOD, r)
            r = jnp.where(r >= MOD, r - MOD, r) + 1
            flat = jnp.where(pos_iota >= n - 1, r, 0) + g * V
            out_ref[g] = flat
            g += 1


def _hash_ids(ids32):
    return pl.pallas_call(
        _hash_body,
        grid=(1,),
        in_specs=[pl.BlockSpec((B, S), lambda i: (_i32(i * 0), _i32(i * 0)))],
        out_specs=pl.BlockSpec(
            (TOTAL, B, S),
            lambda i: (_i32(i * 0), _i32(i * 0), _i32(i * 0))),
        out_shape=jax.ShapeDtypeStruct((TOTAL, B, S), jnp.int32),
    )(ids32)


_NC, _NS = 2, 16
_NW = _NC * _NS
_RPW = ROWS // _NW
_CHUNK = 128
_NCH = _RPW // _CHUNK


def _i32(x):
    return jnp.asarray(x, jnp.int32)


def _gather_body(table_hbm, idx_hbm, out_hbm, idx_v, buf_v, sem0, sem1):
    wid = _i32(lax.axis_index("s") * _NC + lax.axis_index("c"))
    base = wid * _RPW
    pltpu.sync_copy(idx_hbm.at[pl.ds(wid * _NCH, _NCH)], idx_v)
    sems = (sem0, sem1)

    def start(bslot, chunk):
        pltpu.make_async_copy(
            table_hbm.at[idx_v.at[_i32(chunk)]], buf_v.at[_i32(bslot)],
            sems[bslot]
        ).start()

    def finish(bslot, chunk):
        pltpu.make_async_copy(
            table_hbm.at[idx_v.at[_i32(chunk)]], buf_v.at[_i32(bslot)],
            sems[bslot]
        ).wait()
        pltpu.sync_copy(
            buf_v.at[_i32(bslot)],
            out_hbm.at[pl.ds(base + _i32(chunk) * _CHUNK, _CHUNK)])

    for b in range(2):
        start(b, b)

    def body(g2, carry):
        for b in range(2):
            g = g2 * 2 + b
            finish(b, g)
            start(b, g + 2)
        return carry

    lax.fori_loop(_i32(0), _i32((_NCH - 2) // 2), body, _i32(0))
    for b in range(2):
        finish(b, _NCH - 2 + b)


def _sc_gather(table_flat, idx2d):
    mesh = plsc.VectorSubcoreMesh(core_axis_name="c", subcore_axis_name="s")
    k = functools.partial(
        pl.kernel,
        out_type=jax.ShapeDtypeStruct((ROWS, HD), jnp.float32),
        mesh=mesh,
        scratch_types=[
            pltpu.VMEM((_NCH, _CHUNK), jnp.int32),
            pltpu.VMEM((2, _CHUNK, HD), jnp.float32),
            pltpu.SemaphoreType.DMA,
            pltpu.SemaphoreType.DMA,
        ],
    )(_gather_body)
    return k(table_flat, idx2d)


_TILE = 1024
_SJ = S // _TILE


def _dense_body(mem_ref, hid_ref, wkv_ref, knw_ref, vnw_ref, cw_ref,
                out_ref, carry_ref):
    j = pl.program_id(1)
    x = hid_ref[0]
    prev = jnp.where(j > 0, carry_ref[...], 0.0)
    xm1 = jnp.concatenate([prev[1:2], x[:-1]], axis=0)
    xm2 = jnp.concatenate([prev, x[:-2]], axis=0)
    carry_ref[...] = x[_TILE - 2:]
    q = x * cw_ref[2:3] + xm1 * cw_ref[1:2] + xm2 * cw_ref[0:1]

    acc = jnp.zeros((_TILE, 2 * HID), jnp.float32)
    for h in range(TOTAL):
        mh = mem_ref[h]
        w_s = wkv_ref[h * HD:(h + 1) * HD, :]
        acc = acc + lax.dot_general(
            mh, w_s, (((1,), (0,)), ((), ())),
            preferred_element_type=jnp.float32)
    acc_k = acc[:, :HID]
    acc_v = acc[:, HID:]
    vark = jnp.mean(acc_k * acc_k, axis=-1, keepdims=True)
    k_mem = knw_ref[...] * (acc_k * lax.rsqrt(vark + EPS))
    varv = jnp.mean(acc_v * acc_v, axis=-1, keepdims=True)
    v_mem = vnw_ref[...] * (acc_v * lax.rsqrt(varv + EPS))
    logit = (jnp.sum(q * k_mem, axis=-1, keepdims=True) * (1.0 / 32.0)
             + cw_ref[K:K + 1, 0:1])
    gate = jax.nn.sigmoid(logit)
    out_ref[0] = gate * v_mem


def _dense(mem8, hidden, Wkv, knw, vnw, cw):
    grid = (B, _SJ)
    return pl.pallas_call(
        _dense_body,
        grid=grid,
        in_specs=[
            pl.BlockSpec((TOTAL, _TILE, HD),
                         lambda b, j: (_i32(b * 0), _i32(b * _SJ + j),
                                       _i32(b * 0))),
            pl.BlockSpec((1, _TILE, HID),
                         lambda b, j: (_i32(b), _i32(j), _i32(b * 0))),
            pl.BlockSpec((MEMD, 2 * HID),
                         lambda b, j: (_i32(b * 0), _i32(b * 0))),
            pl.BlockSpec((1, HID), lambda b, j: (_i32(b * 0), _i32(b * 0))),
            pl.BlockSpec((1, HID), lambda b, j: (_i32(b * 0), _i32(b * 0))),
            pl.BlockSpec((K + 1, HID),
                         lambda b, j: (_i32(b * 0), _i32(b * 0))),
        ],
        out_specs=pl.BlockSpec((1, _TILE, HID),
                               lambda b, j: (_i32(b), _i32(j), _i32(b * 0))),
        out_shape=jax.ShapeDtypeStruct((B, S, HID), jnp.float32),
        scratch_shapes=[pltpu.VMEM((2, HID), jnp.float32)],
        compiler_params=pltpu.CompilerParams(
            dimension_semantics=("arbitrary", "arbitrary")),
    )(mem8, hidden, Wkv, knw, vnw, cw)


def kernel(hidden_states, input_ids, emb_tables, Wk, Wv, key_norm_w,
           value_norm_w, conv_w, gate_bias):
    ids32 = input_ids.astype(jnp.int32)
    idxs = _hash_ids(ids32)
    idx2d = idxs.reshape(_NW * _NCH, _CHUNK)
    table_flat = emb_tables.reshape(TOTAL * V, HD)
    mem = _sc_gather(table_flat, idx2d)
    mem8 = mem.reshape(TOTAL, POS, HD)
    cwgb = jnp.concatenate(
        [conv_w.reshape(HID, K).T,
         jnp.broadcast_to(jnp.reshape(gate_bias, (1, 1)), (1, HID))], axis=0)
    Wkv = jnp.concatenate([Wk.T, Wv.T], axis=1)
    out = _dense(
        mem8,
        hidden_states,
        Wkv,
        key_norm_w.reshape(1, HID),
        value_norm_w.reshape(1, HID),
        cwgb,
    )
    return out.astype(jnp.float64)


# speedup vs baseline: 1.8058x; 1.0054x over previous
import functools

import jax
import jax.numpy as jnp
from jax import lax
from jax.experimental import pallas as pl
from jax.experimental.pallas import tpu as pltpu
from jax.experimental.pallas import tpu_sc as plsc

B, S, HID = 4, 4096, 1024
TOKEN_VOCAB = 10240
V = 100000
NGRAMS = [2, 3]
NH = 4
TOTAL = len(NGRAMS) * NH
HD = 128
MEMD = TOTAL * HD
K = 3
LAYER_ID = 0
HASH_SEED = 17
MOD = V - 1
EPS = 1e-6
POS = B * S
ROWS = TOTAL * POS

_MAX_INT = (1 << 31) - 1


def _hash_params(n):
    mults, offs = [], []
    for h in range(NH):
        base = HASH_SEED + 10007 * (LAYER_ID + 1) + 1543 * (n + 1) + 8191 * (h + 1)
        hm = []
        for pos in range(n):
            v = (base + 32771 * (pos + 1) + 65537 * (h + 1) * (pos + 1)) % _MAX_INT
            hm.append(v * 2 + 1)
        off = (base * 48271 + 97 * (n + h + 1)) % _MAX_INT
        mults.append(hm)
        offs.append(off)
    return mults, offs


_HP = {n: _hash_params(n) for n in NGRAMS}
_R32 = (1 << 32) % MOD
_R24 = (1 << 24) % MOD


_HR = POS // 128


def _hash_body(ids_ref, out_ref):
    i0 = lax.broadcasted_iota(jnp.int32, (_HR, 128), 0)
    i1 = lax.broadcasted_iota(jnp.int32, (_HR, 128), 1)
    pos_iota = (i0 * 128 + i1) & (S - 1)
    shifted = {0: ids_ref[0], 1: ids_ref[1], 2: ids_ref[2]}
    g = 0
    for n in NGRAMS:
        mults, offs = _HP[n]
        for h in range(NH):
            x0 = jnp.zeros((_HR, 128), jnp.int32)
            x1 = jnp.zeros((_HR, 128), jnp.int32)
            x2 = jnp.zeros((_HR, 128), jnp.int32)
            for p in range(n):
                m = mults[h][p]
                m0 = m & 0xFFFF
                m1 = m >> 16
                idsh = shifted[n - 1 - p]
                a = idsh * m0
                bb = idsh * m1
                a0 = a & 0xFFFF
                a1 = a >> 16
                b0 = bb & 0xFFFF
                b1 = bb >> 16
                l1t = a1 + b0
                c = l1t >> 16
                l1 = l1t & 0xFFFF
                l2 = b1 + c
                x0 = x0 ^ a0
                x1 = x1 ^ l1
                x2 = x2 ^ l2
            off = offs[h]
            o0 = off & 0xFFFF
            o1 = off >> 16
            s0t = x0 + o0
            c0 = s0t >> 16
            s0 = s0t & 0xFFFF
            s1t = x1 + o1 + c0
            c1 = s1t >> 16
            s1 = s1t & 0xFFFF
            s2 = x2 + c1
            tot = s2 * _R32 + (s1 >> 8) * _R24 + (s1 & 0xFF) * 65536 + s0
            q = (tot.astype(jnp.float32) * (1.0 / MOD)).astype(jnp.int32)
            r = tot - q * MOD
            r = jnp.where(r < 0, r + MOD, r)
            r = jnp.where(r >= MOD, r - MOD, r) + 1
            flat = jnp.where(pos_iota >= n - 1, r, 0) + g * V
            out_ref[g] = flat
            g += 1


def _hash_ids(ids3):
    return pl.pallas_call(
        _hash_body,
        grid=(1,),
        in_specs=[pl.BlockSpec(
            (3, _HR, 128),
            lambda i: (_i32(i * 0), _i32(i * 0), _i32(i * 0)))],
        out_specs=pl.BlockSpec(
            (TOTAL, _HR, 128),
            lambda i: (_i32(i * 0), _i32(i * 0), _i32(i * 0))),
        out_shape=jax.ShapeDtypeStruct((TOTAL, _HR, 128), jnp.int32),
        name="ngram_hash",
    )(ids3)


_NC, _NS = 2, 16
_NW = _NC * _NS
_RPW = ROWS // _NW
_CHUNK = 128
_NCH = _RPW // _CHUNK


def _i32(x):
    return jnp.asarray(x, jnp.int32)


def _gather_body(table_hbm, idx_hbm, out_hbm, idx_v, buf_v, sem0, sem1):
    wid = _i32(lax.axis_index("s") * _NC + lax.axis_index("c"))
    base = wid * _RPW
    pltpu.sync_copy(idx_hbm.at[pl.ds(wid * _NCH, _NCH)], idx_v)
    sems = (sem0, sem1)

    def start(bslot, chunk):
        pltpu.make_async_copy(
            table_hbm.at[idx_v.at[_i32(chunk)]], buf_v.at[_i32(bslot)],
            sems[bslot]
        ).start()

    def finish(bslot, chunk):
        pltpu.make_async_copy(
            table_hbm.at[idx_v.at[_i32(chunk)]], buf_v.at[_i32(bslot)],
            sems[bslot]
        ).wait()
        pltpu.sync_copy(
            buf_v.at[_i32(bslot)],
            out_hbm.at[pl.ds(base + _i32(chunk) * _CHUNK, _CHUNK)])

    for b in range(2):
        start(b, b)

    def body(g2, carry):
        for b in range(2):
            g = g2 * 2 + b
            finish(b, g)
            start(b, g + 2)
        return carry

    lax.fori_loop(_i32(0), _i32((_NCH - 2) // 2), body, _i32(0))
    for b in range(2):
        finish(b, _NCH - 2 + b)


def _sc_gather(table_flat, idx2d):
    mesh = plsc.VectorSubcoreMesh(core_axis_name="c", subcore_axis_name="s")
    k = functools.partial(
        pl.kernel,
        out_type=jax.ShapeDtypeStruct((ROWS, HD), jnp.float32),
        mesh=mesh,
        scratch_types=[
            pltpu.VMEM((_NCH, _CHUNK), jnp.int32),
            pltpu.VMEM((2, _CHUNK, HD), jnp.float32),
            pltpu.SemaphoreType.DMA,
            pltpu.SemaphoreType.DMA,
        ],
        name="sc_embed_gather",
    )(_gather_body)
    return k(table_flat, idx2d)


_TILE = 1024
_SJ = S // _TILE


def _dense_body(mem_ref, hid_ref, wkv_ref, knw_ref, vnw_ref, cw_ref,
                out_ref, carry_ref):
    j = pl.program_id(1)
    x = hid_ref[0]
    prev = jnp.where(j > 0, carry_ref[...], 0.0)
    xm1 = jnp.concatenate([prev[1:2], x[:-1]], axis=0)
    xm2 = jnp.concatenate([prev, x[:-2]], axis=0)
    carry_ref[...] = x[_TILE - 2:]
    q = x * cw_ref[2:3] + xm1 * cw_ref[1:2] + xm2 * cw_ref[0:1]

    acc = jnp.zeros((_TILE, 2 * HID), jnp.float32)
    for h in range(TOTAL):
        mh = mem_ref[h].astype(jnp.bfloat16)
        w_s = wkv_ref[h * HD:(h + 1) * HD, :]
        acc = acc + lax.dot_general(
            mh, w_s, (((1,), (0,)), ((), ())),
            preferred_element_type=jnp.float32)
    acc_k = acc[:, :HID]
    acc_v = acc[:, HID:]
    vark = jnp.mean(acc_k * acc_k, axis=-1, keepdims=True)
    k_mem = knw_ref[...] * (acc_k * lax.rsqrt(vark + EPS))
    varv = jnp.mean(acc_v * acc_v, axis=-1, keepdims=True)
    v_mem = vnw_ref[...] * (acc_v * lax.rsqrt(varv + EPS))
    logit = (jnp.sum(q * k_mem, axis=-1, keepdims=True) * (1.0 / 32.0)
             + cw_ref[K:K + 1, 0:1])
    gate = jax.nn.sigmoid(logit)
    out_ref[0] = gate * v_mem


def _dense(mem8, hidden, Wkv, knw, vnw, cw):
    grid = (B, _SJ)
    return pl.pallas_call(
        _dense_body,
        grid=grid,
        in_specs=[
            pl.BlockSpec((TOTAL, _TILE, HD),
                         lambda b, j: (_i32(b * 0), _i32(b * _SJ + j),
                                       _i32(b * 0))),
            pl.BlockSpec((1, _TILE, HID),
                         lambda b, j: (_i32(b), _i32(j), _i32(b * 0))),
            pl.BlockSpec((MEMD, 2 * HID),
                         lambda b, j: (_i32(b * 0), _i32(b * 0))),
            pl.BlockSpec((1, HID), lambda b, j: (_i32(b * 0), _i32(b * 0))),
            pl.BlockSpec((1, HID), lambda b, j: (_i32(b * 0), _i32(b * 0))),
            pl.BlockSpec((K + 1, HID),
                         lambda b, j: (_i32(b * 0), _i32(b * 0))),
        ],
        out_specs=pl.BlockSpec((1, _TILE, HID),
                               lambda b, j: (_i32(b), _i32(j), _i32(b * 0))),
        out_shape=jax.ShapeDtypeStruct((B, S, HID), jnp.float32),
        scratch_shapes=[pltpu.VMEM((2, HID), jnp.float32)],
        compiler_params=pltpu.CompilerParams(
            dimension_semantics=("arbitrary", "arbitrary")),
        name="dense_epilogue",
    )(mem8, hidden, Wkv, knw, vnw, cw)


def kernel(hidden_states, input_ids, emb_tables, Wk, Wv, key_norm_w,
           value_norm_w, conv_w, gate_bias):
    ids32 = input_ids.astype(jnp.int32)
    sh1 = jnp.concatenate(
        [jnp.zeros((B, 1), jnp.int32), ids32[:, :S - 1]], axis=1)
    sh2 = jnp.concatenate(
        [jnp.zeros((B, 2), jnp.int32), ids32[:, :S - 2]], axis=1)
    ids3 = jnp.stack([ids32, sh1, sh2]).reshape(3, _HR, 128)
    idxs = _hash_ids(ids3)
    idx2d = idxs.reshape(_NW * _NCH, _CHUNK)
    table_flat = emb_tables.reshape(TOTAL * V, HD)
    mem = _sc_gather(table_flat, idx2d)
    mem8 = mem.reshape(TOTAL, POS, HD)
    cwgb = jnp.concatenate(
        [conv_w.reshape(HID, K).T,
         jnp.broadcast_to(jnp.reshape(gate_bias, (1, 1)), (1, HID))], axis=0)
    Wkv = jnp.concatenate([Wk.T, Wv.T], axis=1).astype(jnp.bfloat16)
    out = _dense(
        mem8,
        hidden_states,
        Wkv,
        key_norm_w.reshape(1, HID),
        value_norm_w.reshape(1, HID),
        cwgb,
    )
    return out.astype(jnp.float64)
